# baseline (speedup 1.0000x reference)
import functools

import jax
import jax.numpy as jnp
from jax import lax
from jax.experimental import pallas as pl
from jax.experimental.pallas import tpu as pltpu

B = 2048
D = 256
K = 1024
ND = 8
H = D // 2

BM = 256
NBLK = B // BM


def _gelu_exact(x):
    return 0.5 * x * (1.0 + lax.erf(x * (2.0 ** -0.5)))


def _layernorm(x, g, b):
    mu = jnp.mean(x, axis=-1, keepdims=True)
    var = jnp.mean((x - mu) ** 2, axis=-1, keepdims=True)
    return (x - mu) / jnp.sqrt(var + 1e-5) * g + b


def _fused_body(x_ref, did_ref, w1_ref, b1_ref, g1_ref, be1_ref, w2_ref,
                b2_ref, pw1_ref, pb1_ref, pg_ref, pbe_ref, pw2_ref, pb2_ref,
                p_ref, gp_ref, enh_ref, w_ref, pbf_ref, gbf_ref):
    @pl.when(pl.program_id(0) == 0)
    def _():
        pbf_ref[...] = p_ref[...].astype(jnp.bfloat16)
        gbf_ref[...] = gp_ref[...].astype(jnp.bfloat16)

    x = x_ref[...]
    dot = functools.partial(jnp.dot, preferred_element_type=jnp.float32)
    h = dot(x, w1_ref[...]) + b1_ref[...]
    h = _layernorm(h, g1_ref[...], be1_ref[...])
    h = _gelu_exact(h)
    h = dot(h, w2_ref[...]) + b2_ref[...]
    nrm = jnp.sqrt(jnp.sum(h * h, axis=-1, keepdims=True))
    feats = h / jnp.maximum(nrm, 1e-12)
    t = dot(x, pw1_ref[...]) + pb1_ref[...]
    t = _layernorm(t, pg_ref[...], pbe_ref[...])
    t = _gelu_exact(t)
    logits = dot(t, pw2_ref[...]) + pb2_ref[...]
    m = jnp.max(logits, axis=-1, keepdims=True)
    e = jnp.exp(logits - m)
    w = e / jnp.sum(e, axis=-1, keepdims=True)
    w_ref[...] = w
    did = did_ref[0, 0, :].reshape(BM, 1)
    wb = w.astype(jnp.bfloat16)
    acc = feats + 0.08 * dot(wb, gbf_ref[...])
    for n in range(ND):
        sel = (did == n).astype(jnp.float32)
        acc += (0.12 * sel) * dot(wb, pbf_ref[pl.ds(n * K, K), :])
    enh_ref[...] = acc


def kernel(features, domain_ids, ph_W1, ph_b1, ln1_g, ln1_b, ph_W2, ph_b2,
           pw_W1, pw_b1, pw_ln_g, pw_ln_b, pw_W2, pw_b2,
           domain_prototypes, global_prototypes):
    did = jnp.minimum(domain_ids, ND - 1).astype(jnp.int32)
    did3 = did.reshape(NBLK, 1, BM)

    def const(shape):
        return pl.BlockSpec(shape, lambda i: (0,) * len(shape))

    enhanced, w = pl.pallas_call(
        _fused_body,
        grid=(NBLK,),
        in_specs=[
            pl.BlockSpec((BM, D), lambda i: (i, 0)),
            pl.BlockSpec((1, 1, BM), lambda i: (i, 0, 0)),
            const((D, D)), const((1, D)), const((1, D)), const((1, D)),
            const((D, D)), const((1, D)),
            const((D, H)), const((1, H)), const((1, H)), const((1, H)),
            const((H, K)), const((1, K)),
            const((ND * K, D)), const((K, D)),
        ],
        out_specs=[
            pl.BlockSpec((BM, D), lambda i: (i, 0)),
            pl.BlockSpec((BM, K), lambda i: (i, 0)),
        ],
        out_shape=[
            jax.ShapeDtypeStruct((B, D), jnp.float32),
            jax.ShapeDtypeStruct((B, K), jnp.float32),
        ],
        scratch_shapes=[
            pltpu.VMEM((ND * K, D), jnp.bfloat16),
            pltpu.VMEM((K, D), jnp.bfloat16),
        ],
    )(features, did3, ph_W1, ph_b1.reshape(1, D), ln1_g.reshape(1, D),
      ln1_b.reshape(1, D), ph_W2, ph_b2.reshape(1, D),
      pw_W1, pw_b1.reshape(1, H), pw_ln_g.reshape(1, H),
      pw_ln_b.reshape(1, H), pw_W2, pw_b2.reshape(1, K),
      domain_prototypes.reshape(ND * K, D), global_prototypes)
    return (enhanced, w)


# speedup vs baseline: 1.2920x; 1.2920x over previous
import functools

import jax
import jax.numpy as jnp
from jax import lax
from jax.experimental import pallas as pl
from jax.experimental.pallas import tpu as pltpu

B = 2048
D = 256
K = 1024
ND = 8
H = D // 2

BM = 1024
NBLK = B // BM


def _gelu_exact(x):
    return 0.5 * x * (1.0 + lax.erf(x * (2.0 ** -0.5)))


def _layernorm(x, g, b):
    mu = jnp.mean(x, axis=-1, keepdims=True)
    var = jnp.mean((x - mu) ** 2, axis=-1, keepdims=True)
    return (x - mu) / jnp.sqrt(var + 1e-5) * g + b


def _fused_body(x_ref, did_ref, w1_ref, b1_ref, g1_ref, be1_ref, w2_ref,
                b2_ref, pw1_ref, pb1_ref, pg_ref, pbe_ref, pw2_ref, pb2_ref,
                p_ref, gp_ref, enh_ref, w_ref):
    x = x_ref[...]
    dot = functools.partial(jnp.dot, preferred_element_type=jnp.float32)
    h = dot(x, w1_ref[...]) + b1_ref[...]
    h = _layernorm(h, g1_ref[...], be1_ref[...])
    h = _gelu_exact(h)
    h = dot(h, w2_ref[...]) + b2_ref[...]
    nrm = jnp.sqrt(jnp.sum(h * h, axis=-1, keepdims=True))
    feats = h / jnp.maximum(nrm, 1e-12)
    t = dot(x, pw1_ref[...]) + pb1_ref[...]
    t = _layernorm(t, pg_ref[...], pbe_ref[...])
    t = _gelu_exact(t)
    logits = dot(t, pw2_ref[...]) + pb2_ref[...]
    m = jnp.max(logits, axis=-1, keepdims=True)
    e = jnp.exp(logits - m)
    w = e / jnp.sum(e, axis=-1, keepdims=True)
    w_ref[...] = w
    did = did_ref[0, 0, :].reshape(BM, 1)
    acc = feats + 0.08 * dot(w, gp_ref[...])
    for n in range(ND):
        sel = (did == n).astype(jnp.float32)
        acc += (0.12 * sel) * dot(w, p_ref[pl.ds(n * K, K), :])
    enh_ref[...] = acc


def kernel(features, domain_ids, ph_W1, ph_b1, ln1_g, ln1_b, ph_W2, ph_b2,
           pw_W1, pw_b1, pw_ln_g, pw_ln_b, pw_W2, pw_b2,
           domain_prototypes, global_prototypes):
    did = jnp.minimum(domain_ids, ND - 1).astype(jnp.int32)
    did3 = did.reshape(NBLK, 1, BM)

    def const(shape):
        return pl.BlockSpec(shape, lambda i: (0,) * len(shape))

    enhanced, w = pl.pallas_call(
        _fused_body,
        grid=(NBLK,),
        in_specs=[
            pl.BlockSpec((BM, D), lambda i: (i, 0)),
            pl.BlockSpec((1, 1, BM), lambda i: (i, 0, 0)),
            const((D, D)), const((1, D)), const((1, D)), const((1, D)),
            const((D, D)), const((1, D)),
            const((D, H)), const((1, H)), const((1, H)), const((1, H)),
            const((H, K)), const((1, K)),
            const((ND * K, D)), const((K, D)),
        ],
        out_specs=[
            pl.BlockSpec((BM, D), lambda i: (i, 0)),
            pl.BlockSpec((BM, K), lambda i: (i, 0)),
        ],
        out_shape=[
            jax.ShapeDtypeStruct((B, D), jnp.float32),
            jax.ShapeDtypeStruct((B, K), jnp.float32),
        ],
    )(features, did3, ph_W1, ph_b1.reshape(1, D), ln1_g.reshape(1, D),
      ln1_b.reshape(1, D), ph_W2, ph_b2.reshape(1, D),
      pw_W1, pw_b1.reshape(1, H), pw_ln_g.reshape(1, H),
      pw_ln_b.reshape(1, H), pw_W2, pw_b2.reshape(1, K),
      domain_prototypes.reshape(ND * K, D), global_prototypes)
    return (enhanced, w)
